# baseline (speedup 1.0000x reference)
import functools

import jax
import jax.numpy as jnp
from jax import lax
from jax.experimental import pallas as pl
from jax.experimental.pallas import tpu as pltpu
from jax.experimental.pallas import tpu_sc as plsc

K = 512
CAP = 640
CAPP = CAP + 16
HK = K // 2
I32_MIN = -(2 ** 31)
PAD_IDX = 1 << 29
MASK31 = 0x7FFFFFFF


def _monotone_key(bits):
    return bits ^ ((bits >> 31) & jnp.int32(MASK31))


def _select_body(x_ref, pos_ref):
    x = x_ref[...]
    bt, t = x.shape
    b = bt // t
    bits = lax.bitcast_convert_type(x, jnp.int32)
    key = _monotone_key(bits).reshape(b, t, t)

    def cnt_ge(th):
        return jnp.sum((key >= th).astype(jnp.int32), axis=(1, 2),
                       keepdims=True)

    zero = jnp.zeros((b, 1, 1), jnp.int32)
    imin = jnp.full((b, 1, 1), I32_MIN, jnp.int32)
    th = jnp.where(cnt_ge(zero) >= K, zero, imin)
    for bit in range(30, -1, -1):
        trial = th | jnp.int32(1 << bit)
        th = jnp.where(cnt_ge(trial) >= K, trial, th)

    mask = (key >= th).reshape(bt, t)
    mf = mask.astype(jnp.float32)
    r_i = lax.broadcasted_iota(jnp.int32, (t, t), 0)
    c_i = lax.broadcasted_iota(jnp.int32, (t, t), 1)
    upper = (r_i < c_i).astype(jnp.float32)
    excl = jax.lax.dot(mf, upper)
    ones_col = jnp.ones((t, 1), jnp.float32)
    rs_col = jax.lax.dot(mf, ones_col)
    rr = lax.broadcasted_iota(jnp.int32, (bt, bt), 0)
    cc = lax.broadcasted_iota(jnp.int32, (bt, bt), 1)
    blk = jnp.logical_and(cc < rr, (cc // t) == (rr // t)).astype(jnp.float32)
    row_off = jax.lax.dot(blk, rs_col)
    posf = excl + row_off
    posi = posf.astype(jnp.int32)
    pos_ref[...] = jnp.where(mask, posi, jnp.int32(CAP))


def _select(logit2):
    b, n = logit2.shape
    t = 128
    out = pl.pallas_call(
        _select_body,
        out_shape=jax.ShapeDtypeStruct((b * t, t), jnp.int32),
    )(logit2.reshape(b * t, t))
    return out.reshape(b, n)


def _compact(logit2, pos):
    b_total, n = logit2.shape
    mesh = plsc.VectorSubcoreMesh(core_axis_name="c", subcore_axis_name="s")

    @functools.partial(
        pl.kernel,
        out_type=[jax.ShapeDtypeStruct((b_total, CAP), jnp.float32),
                  jax.ShapeDtypeStruct((b_total, CAP), jnp.int32)],
        mesh=mesh,
        compiler_params=pltpu.CompilerParams(needs_layout_passes=False),
        scratch_types=[
            pltpu.VMEM((n,), jnp.float32),
            pltpu.VMEM((n,), jnp.int32),
            pltpu.VMEM((CAPP,), jnp.float32),
            pltpu.VMEM((CAPP,), jnp.int32),
            pltpu.SemaphoreType.DMA,
        ],
    )
    def run(logit_hbm, pos_hbm, ckey_hbm, cidx_hbm, vals_v, pos_v, ckey_v,
            cidx_v, sem):
        c = lax.axis_index("c")
        s = lax.axis_index("s")

        @pl.when(c == 0)
        def _():
            b = s
            cp = pltpu.async_copy(pos_hbm.at[b], pos_v, sem)
            pltpu.sync_copy(logit_hbm.at[b], vals_v)
            cp.wait()
            lane = lax.broadcasted_iota(jnp.int32, (16,), 0)

            def pre(i, carry):
                ckey_v[pl.ds(i * 16, 16)] = jnp.full((16,), -jnp.inf,
                                                     jnp.float32)
                cidx_v[pl.ds(i * 16, 16)] = jnp.full((16,), PAD_IDX,
                                                     jnp.int32)
                return carry

            lax.fori_loop(0, CAPP // 16, pre, 0)

            def body(i, carry):
                v = vals_v[pl.ds(i * 16, 16)]
                p = pos_v[pl.ds(i * 16, 16)]
                m = p < CAP
                plsc.store_scatter(ckey_v, [p], v, mask=m)
                plsc.store_scatter(cidx_v, [p], lane + i * 16, mask=m)
                return carry

            lax.fori_loop(0, n // 16, body, 0)
            pltpu.sync_copy(ckey_v.at[pl.ds(0, CAP)], ckey_hbm.at[b])
            pltpu.sync_copy(cidx_v.at[pl.ds(0, CAP)], cidx_hbm.at[b])

    return run(logit2, pos)


def _rank_body(kr_ref, kc_ref, ir_ref, ic_ref, out_ref):
    krow = kr_ref[0]
    kcol = kc_ref[0]
    irow = ir_ref[0]
    icol = ic_ref[0]
    gt = kcol > krow
    tie = jnp.logical_and(kcol == krow, icol < irow)
    cnt = jnp.sum(jnp.logical_or(gt, tie).astype(jnp.int32), axis=0,
                  keepdims=True)
    out_ref[0] = cnt


def _rank(ckey, cidx):
    b = ckey.shape[0]
    kr = ckey.reshape(b, 1, CAP)
    kc = ckey.reshape(b, CAP, 1)
    ir = cidx.reshape(b, 1, CAP)
    ic = cidx.reshape(b, CAP, 1)
    row_spec = pl.BlockSpec((1, 1, CAP), lambda i: (i, 0, 0))
    col_spec = pl.BlockSpec((1, CAP, 1), lambda i: (i, 0, 0))
    rank3 = pl.pallas_call(
        _rank_body,
        grid=(b,),
        in_specs=[row_spec, col_spec, row_spec, col_spec],
        out_specs=row_spec,
        out_shape=jax.ShapeDtypeStruct((b, 1, CAP), jnp.int32),
    )(kr, kc, ir, ic)
    return rank3.reshape(b, CAP)


def _gather_all(cidx, rank, map_flat, off_rows, tmap_rows):
    b_total = cidx.shape[0]
    n = map_flat.shape[0] // b_total
    d = map_flat.shape[1]
    nm1 = b_total * n - 1
    mesh = plsc.VectorSubcoreMesh(core_axis_name="c", subcore_axis_name="s")

    @functools.partial(
        pl.kernel,
        out_type=[jax.ShapeDtypeStruct((b_total, K, d), jnp.float32),
                  jax.ShapeDtypeStruct((b_total, K, 2), jnp.int32),
                  jax.ShapeDtypeStruct((b_total, K, 2), jnp.float32),
                  jax.ShapeDtypeStruct((b_total, K), jnp.float32)],
        mesh=mesh,
        compiler_params=pltpu.CompilerParams(needs_layout_passes=False),
        scratch_types=[
            pltpu.VMEM((CAP,), jnp.int32),
            pltpu.VMEM((CAP,), jnp.int32),
            pltpu.VMEM((4, 64), jnp.int32),
            pltpu.VMEM((HK,), jnp.int32),
            pltpu.VMEM((2, 128), jnp.int32),
            pltpu.VMEM((2, 128), jnp.int32),
            pltpu.VMEM((64, d), jnp.float32),
            pltpu.VMEM((128, 128), jnp.float32),
            pltpu.VMEM((HK, 2), jnp.float32),
            pltpu.VMEM((HK,), jnp.float32),
            pltpu.VMEM((HK, 2), jnp.int32),
            pltpu.SemaphoreType.DMA,
        ],
    )
    def run(cidx_hbm, rank_hbm, map_hbm, offr_hbm, tmapr_hbm,
            prop_hbm, pred_hbm, off_hbm, score_hbm,
            cidx_v, rank_v, gmap_v, gflat_v, orow_v, trow_v,
            rows_map, rows_sm, off_v, sc_v, pred_v, sem):
        c = lax.axis_index("c")
        s = lax.axis_index("s")
        b = s
        lo = c * HK
        pltpu.sync_copy(cidx_hbm.at[b], cidx_v)
        pltpu.sync_copy(rank_hbm.at[b], rank_v)
        lane = lax.broadcasted_iota(jnp.int32, (16,), 0)
        zeros16 = jnp.zeros((16,), jnp.int32)
        ones16 = jnp.full((16,), 1, jnp.int32)

        def sbody(i, carry):
            rk = rank_v[pl.ds(i * 16, 16)]
            ix = cidx_v[pl.ds(i * 16, 16)]
            rrel = rk - lo
            m = jnp.logical_and(rrel >= 0, rrel < HK)
            rsafe = rrel & (HK - 1)
            g = (ix + b * n) & nm1
            plsc.store_scatter(gmap_v, [rsafe >> 6, rsafe & 63], g, mask=m)
            plsc.store_scatter(gflat_v, [rsafe], g, mask=m)
            plsc.store_scatter(orow_v, [rsafe >> 7, rsafe & 127], g >> 6,
                               mask=m)
            plsc.store_scatter(trow_v, [rsafe >> 7, rsafe & 127], g >> 7,
                               mask=m)
            return carry

        lax.fori_loop(0, CAP // 16, sbody, 0)

        for j in range(4):
            pltpu.async_copy(map_hbm.at[gmap_v.at[j]], rows_map, sem).wait()
            pltpu.sync_copy(rows_map, prop_hbm.at[b, pl.ds(lo + j * 64, 64)])

        for tchunk in range(2):
            pltpu.async_copy(offr_hbm.at[orow_v.at[tchunk]], rows_sm,
                             sem).wait()
            for i in range(8):
                base = tchunk * 128 + i * 16
                g16 = gflat_v[pl.ds(base, 16)]
                j16 = lane + i * 16
                col = (g16 * 2) & 127
                o0 = plsc.load_gather(rows_sm, [j16, col])
                o1 = plsc.load_gather(rows_sm, [j16, col + 1])
                ridx = j16 + tchunk * 128
                plsc.store_scatter(off_v, [ridx, zeros16], o0)
                plsc.store_scatter(off_v, [ridx, ones16], o1)

        for tchunk in range(2):
            pltpu.async_copy(tmapr_hbm.at[trow_v.at[tchunk]], rows_sm,
                             sem).wait()
            for i in range(8):
                base = tchunk * 128 + i * 16
                g16 = gflat_v[pl.ds(base, 16)]
                j16 = lane + i * 16
                sc_v[pl.ds(base, 16)] = plsc.load_gather(
                    rows_sm, [j16, g16 & 127])

        for i in range(HK // 16):
            g16 = gflat_v[pl.ds(i * 16, 16)]
            ii = g16 - b * n
            ridx = lane + i * 16
            plsc.store_scatter(pred_v, [ridx, zeros16], ii >> 7)
            plsc.store_scatter(pred_v, [ridx, ones16], (ii & 127) + 1)

        out_slice = pl.ds(lo, HK)
        pltpu.sync_copy(pred_v, pred_hbm.at[b, out_slice])
        pltpu.sync_copy(off_v, off_hbm.at[b, out_slice])
        pltpu.sync_copy(sc_v, score_hbm.at[b, out_slice])

    return run(cidx, rank, map_flat, off_rows, tmap_rows)


@jax.jit
def kernel(selection_logit, map2d, offset_gt, tmap):
    b, t, _ = selection_logit.shape
    n = t * t
    d = map2d.shape[-1]
    logit2 = selection_logit.reshape(b, n)
    pos = _select(logit2)
    ckey, cidx = _compact(logit2, pos)
    rank = _rank(ckey, cidx)
    return tuple(_gather_all(cidx, rank,
                             map2d.reshape(b * n, d),
                             offset_gt.reshape(b * n * 2 // 128, 128),
                             tmap.reshape(b * n // 128, 128)))


# speedup vs baseline: 1.0015x; 1.0015x over previous
import functools

import jax
import jax.numpy as jnp
from jax import lax
from jax.experimental import pallas as pl
from jax.experimental.pallas import tpu as pltpu
from jax.experimental.pallas import tpu_sc as plsc

K = 512
CAP = 640
CAPP = CAP + 16
HK = K // 2
I32_MIN = -(2 ** 31)
PAD_IDX = 1 << 29
MASK31 = 0x7FFFFFFF


def _monotone_key(bits):
    return bits ^ ((bits >> 31) & jnp.int32(MASK31))


def _select_body(x_ref, pos_ref):
    x = x_ref[...]
    bt, t = x.shape
    b = bt // t
    bits = lax.bitcast_convert_type(x, jnp.int32)
    key = _monotone_key(bits).reshape(b, t, t)

    def cnt_ge(th):
        return jnp.sum((key >= th).astype(jnp.int32), axis=(1, 2),
                       keepdims=True)

    zero = jnp.zeros((b, 1, 1), jnp.int32)
    imin = jnp.full((b, 1, 1), I32_MIN, jnp.int32)
    th = jnp.where(cnt_ge(zero) >= K, zero, imin)
    for bit in range(30, -1, -1):
        trial = th | jnp.int32(1 << bit)
        th = jnp.where(cnt_ge(trial) >= K, trial, th)

    mask = (key >= th).reshape(bt, t)
    mf = mask.astype(jnp.float32)
    r_i = lax.broadcasted_iota(jnp.int32, (t, t), 0)
    c_i = lax.broadcasted_iota(jnp.int32, (t, t), 1)
    upper = (r_i < c_i).astype(jnp.float32)
    excl = jax.lax.dot(mf, upper)
    ones_col = jnp.ones((t, 1), jnp.float32)
    rs_col = jax.lax.dot(mf, ones_col)
    rr = lax.broadcasted_iota(jnp.int32, (bt, bt), 0)
    cc = lax.broadcasted_iota(jnp.int32, (bt, bt), 1)
    blk = jnp.logical_and(cc < rr, (cc // t) == (rr // t)).astype(jnp.float32)
    row_off = jax.lax.dot(blk, rs_col)
    posf = excl + row_off
    posi = posf.astype(jnp.int32)
    pos_ref[...] = jnp.where(mask, posi, jnp.int32(CAP))


def _select(logit2):
    b, n = logit2.shape
    t = 128
    out = pl.pallas_call(
        _select_body,
        out_shape=jax.ShapeDtypeStruct((b * t, t), jnp.int32),
    )(logit2.reshape(b * t, t))
    return out.reshape(b, n)


def _compact(logit2, pos):
    b_total, n = logit2.shape
    mesh = plsc.VectorSubcoreMesh(core_axis_name="c", subcore_axis_name="s")

    @functools.partial(
        pl.kernel,
        out_type=[jax.ShapeDtypeStruct((b_total, CAP), jnp.float32),
                  jax.ShapeDtypeStruct((b_total, CAP), jnp.int32)],
        mesh=mesh,
        compiler_params=pltpu.CompilerParams(needs_layout_passes=False),
        scratch_types=[
            pltpu.VMEM((n,), jnp.float32),
            pltpu.VMEM((n,), jnp.int32),
            pltpu.VMEM((CAPP,), jnp.float32),
            pltpu.VMEM((CAPP,), jnp.int32),
            pltpu.SemaphoreType.DMA,
        ],
    )
    def run(logit_hbm, pos_hbm, ckey_hbm, cidx_hbm, vals_v, pos_v, ckey_v,
            cidx_v, sem):
        c = lax.axis_index("c")
        s = lax.axis_index("s")

        @pl.when(c == 0)
        def _():
            b = s
            cp = pltpu.async_copy(pos_hbm.at[b], pos_v, sem)
            pltpu.sync_copy(logit_hbm.at[b], vals_v)
            cp.wait()
            lane = lax.broadcasted_iota(jnp.int32, (16,), 0)

            def pre(i, carry):
                ckey_v[pl.ds(i * 16, 16)] = jnp.full((16,), -jnp.inf,
                                                     jnp.float32)
                cidx_v[pl.ds(i * 16, 16)] = jnp.full((16,), PAD_IDX,
                                                     jnp.int32)
                return carry

            lax.fori_loop(0, CAPP // 16, pre, 0)

            def body(i, carry):
                v = vals_v[pl.ds(i * 16, 16)]
                p = pos_v[pl.ds(i * 16, 16)]
                m = p < CAP
                plsc.store_scatter(ckey_v, [p], v, mask=m)
                plsc.store_scatter(cidx_v, [p], lane + i * 16, mask=m)
                return carry

            lax.fori_loop(0, n // 16, body, 0)
            pltpu.sync_copy(ckey_v.at[pl.ds(0, CAP)], ckey_hbm.at[b])
            pltpu.sync_copy(cidx_v.at[pl.ds(0, CAP)], cidx_hbm.at[b])

    return run(logit2, pos)


def _rank_body(kr_ref, kc_ref, ir_ref, ic_ref, out_ref):
    krow = kr_ref[0]
    kcol = kc_ref[0]
    irow = ir_ref[0]
    icol = ic_ref[0]
    gt = kcol > krow
    tie = jnp.logical_and(kcol == krow, icol < irow)
    cnt = jnp.sum(jnp.logical_or(gt, tie).astype(jnp.int32), axis=0,
                  keepdims=True)
    out_ref[0] = cnt


def _rank(ckey, cidx):
    b = ckey.shape[0]
    kr = ckey.reshape(b, 1, CAP)
    kc = ckey.reshape(b, CAP, 1)
    ir = cidx.reshape(b, 1, CAP)
    ic = cidx.reshape(b, CAP, 1)
    row_spec = pl.BlockSpec((1, 1, CAP), lambda i: (i, 0, 0))
    col_spec = pl.BlockSpec((1, CAP, 1), lambda i: (i, 0, 0))
    rank3 = pl.pallas_call(
        _rank_body,
        grid=(b,),
        in_specs=[row_spec, col_spec, row_spec, col_spec],
        out_specs=row_spec,
        out_shape=jax.ShapeDtypeStruct((b, 1, CAP), jnp.int32),
    )(kr, kc, ir, ic)
    return rank3.reshape(b, CAP)


def _gather_all(cidx, rank, map_flat, off_rows, tmap_rows):
    b_total = cidx.shape[0]
    n = map_flat.shape[0] // b_total
    d = map_flat.shape[1]
    nm1 = b_total * n - 1
    mesh = plsc.VectorSubcoreMesh(core_axis_name="c", subcore_axis_name="s")

    @functools.partial(
        pl.kernel,
        out_type=[jax.ShapeDtypeStruct((b_total, K, d), jnp.float32),
                  jax.ShapeDtypeStruct((b_total, K, 2), jnp.int32),
                  jax.ShapeDtypeStruct((b_total, K, 2), jnp.float32),
                  jax.ShapeDtypeStruct((b_total, K), jnp.float32)],
        mesh=mesh,
        compiler_params=pltpu.CompilerParams(needs_layout_passes=False,
                                             use_tc_tiling_on_sc=True),
        scratch_types=[
            pltpu.VMEM((CAP,), jnp.int32),
            pltpu.VMEM((CAP,), jnp.int32),
            pltpu.VMEM((4, 64), jnp.int32),
            pltpu.VMEM((HK,), jnp.int32),
            pltpu.VMEM((2, 128), jnp.int32),
            pltpu.VMEM((2, 128), jnp.int32),
            pltpu.VMEM((64, d), jnp.float32),
            pltpu.VMEM((128, 128), jnp.float32),
            pltpu.VMEM((HK, 2), jnp.float32),
            pltpu.VMEM((HK,), jnp.float32),
            pltpu.VMEM((HK, 2), jnp.int32),
            pltpu.SemaphoreType.DMA,
        ],
    )
    def run(cidx_hbm, rank_hbm, map_hbm, offr_hbm, tmapr_hbm,
            prop_hbm, pred_hbm, off_hbm, score_hbm,
            cidx_v, rank_v, gmap_v, gflat_v, orow_v, trow_v,
            rows_map, rows_sm, off_v, sc_v, pred_v, sem):
        c = lax.axis_index("c")
        s = lax.axis_index("s")
        b = s
        lo = c * HK
        pltpu.sync_copy(cidx_hbm.at[b], cidx_v)
        pltpu.sync_copy(rank_hbm.at[b], rank_v)
        lane = lax.broadcasted_iota(jnp.int32, (16,), 0)
        zeros16 = jnp.zeros((16,), jnp.int32)
        ones16 = jnp.full((16,), 1, jnp.int32)

        def sbody(i, carry):
            rk = rank_v[pl.ds(i * 16, 16)]
            ix = cidx_v[pl.ds(i * 16, 16)]
            rrel = rk - lo
            m = jnp.logical_and(rrel >= 0, rrel < HK)
            rsafe = rrel & (HK - 1)
            g = (ix + b * n) & nm1
            plsc.store_scatter(gmap_v, [rsafe >> 6, rsafe & 63], g, mask=m)
            plsc.store_scatter(gflat_v, [rsafe], g, mask=m)
            plsc.store_scatter(orow_v, [rsafe >> 7, rsafe & 127], g >> 6,
                               mask=m)
            plsc.store_scatter(trow_v, [rsafe >> 7, rsafe & 127], g >> 7,
                               mask=m)
            return carry

        lax.fori_loop(0, CAP // 16, sbody, 0)

        for j in range(4):
            pltpu.async_copy(map_hbm.at[gmap_v.at[j]], rows_map, sem).wait()
            pltpu.sync_copy(rows_map, prop_hbm.at[b, pl.ds(lo + j * 64, 64)])

        for tchunk in range(2):
            pltpu.async_copy(offr_hbm.at[orow_v.at[tchunk]], rows_sm,
                             sem).wait()
            for i in range(8):
                base = tchunk * 128 + i * 16
                g16 = gflat_v[pl.ds(base, 16)]
                j16 = lane + i * 16
                col = (g16 * 2) & 127
                o0 = plsc.load_gather(rows_sm, [j16, col])
                o1 = plsc.load_gather(rows_sm, [j16, col + 1])
                ridx = j16 + tchunk * 128
                plsc.store_scatter(off_v, [ridx, zeros16], o0)
                plsc.store_scatter(off_v, [ridx, ones16], o1)

        for tchunk in range(2):
            pltpu.async_copy(tmapr_hbm.at[trow_v.at[tchunk]], rows_sm,
                             sem).wait()
            for i in range(8):
                base = tchunk * 128 + i * 16
                g16 = gflat_v[pl.ds(base, 16)]
                j16 = lane + i * 16
                sc_v[pl.ds(base, 16)] = plsc.load_gather(
                    rows_sm, [j16, g16 & 127])

        for i in range(HK // 16):
            g16 = gflat_v[pl.ds(i * 16, 16)]
            ii = g16 - b * n
            ridx = lane + i * 16
            plsc.store_scatter(pred_v, [ridx, zeros16], ii >> 7)
            plsc.store_scatter(pred_v, [ridx, ones16], (ii & 127) + 1)

        out_slice = pl.ds(lo, HK)
        pltpu.sync_copy(pred_v, pred_hbm.at[b, out_slice])
        pltpu.sync_copy(off_v, off_hbm.at[b, out_slice])
        pltpu.sync_copy(sc_v, score_hbm.at[b, out_slice])

    return run(cidx, rank, map_flat, off_rows, tmap_rows)


@jax.jit
def kernel(selection_logit, map2d, offset_gt, tmap):
    b, t, _ = selection_logit.shape
    n = t * t
    d = map2d.shape[-1]
    logit2 = selection_logit.reshape(b, n)
    pos = _select(logit2)
    ckey, cidx = _compact(logit2, pos)
    rank = _rank(ckey, cidx)
    return tuple(_gather_all(cidx, rank,
                             map2d.reshape(b * n, d),
                             offset_gt.reshape(b * n * 2 // 128, 128),
                             tmap.reshape(b * n // 128, 128)))


# speedup vs baseline: 2.3713x; 2.3677x over previous
import functools

import jax
import jax.numpy as jnp
from jax import lax
from jax.experimental import pallas as pl
from jax.experimental.pallas import tpu as pltpu
from jax.experimental.pallas import tpu_sc as plsc

K = 512
CAP = 640
CAPP = CAP + 16
HK = K // 2
I32_MIN = -(2 ** 31)
PAD_IDX = 1 << 29
MASK31 = 0x7FFFFFFF


def _monotone_key(bits):
    return bits ^ ((bits >> 31) & jnp.int32(MASK31))


def _select_body(x_ref, pos_ref):
    x = x_ref[...]
    bt, t = x.shape
    b = bt // t
    bits = lax.bitcast_convert_type(x, jnp.int32)
    key = _monotone_key(bits).reshape(b, t, t)

    def cnt_ge(th):
        return jnp.sum((key >= th).astype(jnp.int32), axis=(1, 2),
                       keepdims=True)

    zero = jnp.zeros((b, 1, 1), jnp.int32)
    imin = jnp.full((b, 1, 1), I32_MIN, jnp.int32)
    th = jnp.where(cnt_ge(zero) >= K, zero, imin)
    for bit in range(30, -1, -1):
        trial = th | jnp.int32(1 << bit)
        th = jnp.where(cnt_ge(trial) >= K, trial, th)

    mask = (key >= th).reshape(bt, t)
    mf = mask.astype(jnp.float32)
    r_i = lax.broadcasted_iota(jnp.int32, (t, t), 0)
    c_i = lax.broadcasted_iota(jnp.int32, (t, t), 1)
    upper = (r_i < c_i).astype(jnp.float32)
    excl = jax.lax.dot(mf, upper)
    ones_col = jnp.ones((t, 1), jnp.float32)
    rs_col = jax.lax.dot(mf, ones_col)
    rr = lax.broadcasted_iota(jnp.int32, (bt, bt), 0)
    cc = lax.broadcasted_iota(jnp.int32, (bt, bt), 1)
    blk = jnp.logical_and(cc < rr, (cc // t) == (rr // t)).astype(jnp.float32)
    row_off = jax.lax.dot(blk, rs_col)
    posf = excl + row_off
    posi = posf.astype(jnp.int32)
    pos_ref[...] = jnp.where(mask, posi, jnp.int32(CAP))


def _select(logit2):
    b, n = logit2.shape
    t = 128
    out = pl.pallas_call(
        _select_body,
        out_shape=jax.ShapeDtypeStruct((b * t, t), jnp.int32),
    )(logit2.reshape(b * t, t))
    return out.reshape(b, n)


def _compact(logit2, pos):
    b_total, n = logit2.shape
    mesh = plsc.VectorSubcoreMesh(core_axis_name="c", subcore_axis_name="s")

    @functools.partial(
        pl.kernel,
        out_type=[jax.ShapeDtypeStruct((b_total, CAP), jnp.float32),
                  jax.ShapeDtypeStruct((b_total, CAP), jnp.int32)],
        mesh=mesh,
        compiler_params=pltpu.CompilerParams(needs_layout_passes=False),
        scratch_types=[
            pltpu.VMEM((n,), jnp.float32),
            pltpu.VMEM((n,), jnp.int32),
            pltpu.VMEM((CAPP,), jnp.float32),
            pltpu.VMEM((CAPP,), jnp.int32),
            pltpu.SemaphoreType.DMA,
        ],
    )
    def run(logit_hbm, pos_hbm, ckey_hbm, cidx_hbm, vals_v, pos_v, ckey_v,
            cidx_v, sem):
        c = lax.axis_index("c")
        s = lax.axis_index("s")

        @pl.when(c == 0)
        def _():
            b = s
            cp = pltpu.async_copy(pos_hbm.at[b], pos_v, sem)
            pltpu.sync_copy(logit_hbm.at[b], vals_v)
            cp.wait()
            lane = lax.broadcasted_iota(jnp.int32, (16,), 0)

            def pre(i, carry):
                ckey_v[pl.ds(i * 16, 16)] = jnp.full((16,), -jnp.inf,
                                                     jnp.float32)
                cidx_v[pl.ds(i * 16, 16)] = jnp.full((16,), PAD_IDX,
                                                     jnp.int32)
                return carry

            lax.fori_loop(0, CAPP // 16, pre, 0)

            def body(i, carry):
                v = vals_v[pl.ds(i * 16, 16)]
                p = pos_v[pl.ds(i * 16, 16)]
                m = p < CAP
                plsc.store_scatter(ckey_v, [p], v, mask=m)
                plsc.store_scatter(cidx_v, [p], lane + i * 16, mask=m)
                return carry

            lax.fori_loop(0, n // 16, body, 0)
            pltpu.sync_copy(ckey_v.at[pl.ds(0, CAP)], ckey_hbm.at[b])
            pltpu.sync_copy(cidx_v.at[pl.ds(0, CAP)], cidx_hbm.at[b])

    return run(logit2, pos)


def _rank_body(kr_ref, kc_ref, ir_ref, ic_ref, out_ref):
    krow = kr_ref[0]
    kcol = kc_ref[0]
    irow = ir_ref[0]
    icol = ic_ref[0]
    gt = kcol > krow
    tie = jnp.logical_and(kcol == krow, icol < irow)
    cnt = jnp.sum(jnp.logical_or(gt, tie).astype(jnp.int32), axis=0,
                  keepdims=True)
    out_ref[0] = cnt


def _rank(ckey, cidx):
    b = ckey.shape[0]
    kr = ckey.reshape(b, 1, CAP)
    kc = ckey.reshape(b, CAP, 1)
    ir = cidx.reshape(b, 1, CAP)
    ic = cidx.reshape(b, CAP, 1)
    row_spec = pl.BlockSpec((1, 1, CAP), lambda i: (i, 0, 0))
    col_spec = pl.BlockSpec((1, CAP, 1), lambda i: (i, 0, 0))
    rank3 = pl.pallas_call(
        _rank_body,
        grid=(b,),
        in_specs=[row_spec, col_spec, row_spec, col_spec],
        out_specs=row_spec,
        out_shape=jax.ShapeDtypeStruct((b, 1, CAP), jnp.int32),
    )(kr, kc, ir, ic)
    return rank3.reshape(b, CAP)


def _gather_all(cidx, rank, map_flat, off_rows, tmap_rows):
    b_total = cidx.shape[0]
    n = map_flat.shape[0] // b_total
    d = map_flat.shape[1]
    nm1 = b_total * n - 1
    mesh = plsc.VectorSubcoreMesh(core_axis_name="c", subcore_axis_name="s")

    @functools.partial(
        pl.kernel,
        out_type=[jax.ShapeDtypeStruct((b_total, K, d), jnp.float32),
                  jax.ShapeDtypeStruct((b_total, K, 2), jnp.int32),
                  jax.ShapeDtypeStruct((b_total, K, 2), jnp.float32),
                  jax.ShapeDtypeStruct((b_total, K), jnp.float32)],
        mesh=mesh,
        compiler_params=pltpu.CompilerParams(needs_layout_passes=False,
                                             use_tc_tiling_on_sc=True),
        scratch_types=[
            pltpu.VMEM((CAP,), jnp.int32),
            pltpu.VMEM((CAP,), jnp.int32),
            pltpu.VMEM((4, 64), jnp.int32),
            pltpu.VMEM((HK,), jnp.int32),
            pltpu.VMEM((4, 128), jnp.int32),
            pltpu.VMEM((2, 128), jnp.int32),
            pltpu.VMEM((64, d), jnp.float32),
            pltpu.VMEM((128, 128), jnp.float32),
            pltpu.VMEM((HK, 2), jnp.float32),
            pltpu.VMEM((HK,), jnp.float32),
            pltpu.VMEM((HK, 2), jnp.int32),
            pltpu.SemaphoreType.DMA,
        ],
    )
    def run(cidx_hbm, rank_hbm, map_hbm, offr_hbm, tmapr_hbm,
            prop_hbm, pred_hbm, off_hbm, score_hbm,
            cidx_v, rank_v, gmap_v, gflat_v, orow_v, trow_v,
            rows_map, rows_sm, off_v, sc_v, pred_v, sem):
        c = lax.axis_index("c")
        s = lax.axis_index("s")
        b = s
        lo = c * HK
        pltpu.sync_copy(cidx_hbm.at[b], cidx_v)
        pltpu.sync_copy(rank_hbm.at[b], rank_v)
        lane = lax.broadcasted_iota(jnp.int32, (16,), 0)
        zeros16 = jnp.zeros((16,), jnp.int32)
        ones16 = jnp.full((16,), 1, jnp.int32)

        def sbody(i, carry):
            rk = rank_v[pl.ds(i * 16, 16)]
            ix = cidx_v[pl.ds(i * 16, 16)]
            rrel = rk - lo
            m = jnp.logical_and(rrel >= 0, rrel < HK)
            rsafe = rrel & (HK - 1)
            g = (ix + b * n) & nm1
            plsc.store_scatter(gmap_v, [rsafe >> 6, rsafe & 63], g, mask=m)
            plsc.store_scatter(gflat_v, [rsafe], g, mask=m)
            jl2 = (rsafe & 63) * 2
            plsc.store_scatter(orow_v, [rsafe >> 6, jl2], (g >> 7) * 2,
                               mask=m)
            plsc.store_scatter(orow_v, [rsafe >> 6, jl2 + 1],
                               (g >> 7) * 2 + 1, mask=m)
            plsc.store_scatter(trow_v, [rsafe >> 7, rsafe & 127], g >> 7,
                               mask=m)
            return carry

        lax.fori_loop(0, CAP // 16, sbody, 0)

        for j in range(4):
            pltpu.async_copy(map_hbm.at[gmap_v.at[j]], rows_map, sem).wait()
            pltpu.sync_copy(rows_map, prop_hbm.at[b, pl.ds(lo + j * 64, 64)])

        for tchunk in range(4):
            pltpu.async_copy(offr_hbm.at[orow_v.at[tchunk]], rows_sm,
                             sem).wait()
            for i in range(4):
                base = tchunk * 64 + i * 16
                g16 = gflat_v[pl.ds(base, 16)]
                j16 = lane + i * 16
                jl2 = ((j16 * 2) & 127)
                col = g16 & 127
                o0 = plsc.load_gather(rows_sm, [jl2, col])
                o1 = plsc.load_gather(rows_sm, [jl2 + 1, col])
                ridx = j16 + tchunk * 64
                plsc.store_scatter(off_v, [ridx, zeros16], o0)
                plsc.store_scatter(off_v, [ridx, ones16], o1)

        for tchunk in range(2):
            pltpu.async_copy(tmapr_hbm.at[trow_v.at[tchunk]], rows_sm,
                             sem).wait()
            for i in range(8):
                base = tchunk * 128 + i * 16
                g16 = gflat_v[pl.ds(base, 16)]
                j16 = lane + i * 16
                sc_v[pl.ds(base, 16)] = plsc.load_gather(
                    rows_sm, [j16, g16 & 127])

        for i in range(HK // 16):
            g16 = gflat_v[pl.ds(i * 16, 16)]
            ii = g16 - b * n
            ridx = lane + i * 16
            plsc.store_scatter(pred_v, [ridx, zeros16], ii >> 7)
            plsc.store_scatter(pred_v, [ridx, ones16], (ii & 127) + 1)

        out_slice = pl.ds(lo, HK)
        pltpu.sync_copy(pred_v, pred_hbm.at[b, out_slice])
        pltpu.sync_copy(off_v, off_hbm.at[b, out_slice])
        pltpu.sync_copy(sc_v, score_hbm.at[b, out_slice])

    return run(cidx, rank, map_flat, off_rows, tmap_rows)


@jax.jit
def kernel(selection_logit, map2d, offset_gt, tmap):
    b, t, _ = selection_logit.shape
    n = t * t
    d = map2d.shape[-1]
    logit2 = selection_logit.reshape(b, n)
    pos = _select(logit2)
    ckey, cidx = _compact(logit2, pos)
    rank = _rank(ckey, cidx)
    off_rows = offset_gt.transpose(0, 1, 3, 2).reshape(b * t * 2, t)
    return tuple(_gather_all(cidx, rank,
                             map2d.reshape(b * n, d),
                             off_rows,
                             tmap.reshape(b * n // 128, 128)))


# speedup vs baseline: 3.0124x; 1.2704x over previous
import functools

import jax
import jax.numpy as jnp
from jax import lax
from jax.experimental import pallas as pl
from jax.experimental.pallas import tpu as pltpu
from jax.experimental.pallas import tpu_sc as plsc

K = 512
CAP = 640
CAPP = CAP + 16
HK = K // 2
I32_MIN = -(2 ** 31)
PAD_IDX = 1 << 29
MASK31 = 0x7FFFFFFF


def _monotone_key(bits):
    return bits ^ ((bits >> 31) & jnp.int32(MASK31))


def _select_body(x_ref, pos_ref):
    x = x_ref[...]
    bt, t = x.shape
    b = bt // t
    bits = lax.bitcast_convert_type(x, jnp.int32)
    key = _monotone_key(bits).reshape(b, t, t)

    def cnt_ge(th):
        return jnp.sum((key >= th).astype(jnp.int32), axis=(1, 2),
                       keepdims=True)

    zero = jnp.zeros((b, 1, 1), jnp.int32)
    imin = jnp.full((b, 1, 1), I32_MIN, jnp.int32)
    th = jnp.where(cnt_ge(zero) >= K, zero, imin)
    for bit in range(30, -1, -1):
        trial = th | jnp.int32(1 << bit)
        th = jnp.where(cnt_ge(trial) >= K, trial, th)

    mask = (key >= th).reshape(bt, t)
    mf = mask.astype(jnp.float32)
    r_i = lax.broadcasted_iota(jnp.int32, (t, t), 0)
    c_i = lax.broadcasted_iota(jnp.int32, (t, t), 1)
    upper = (r_i < c_i).astype(jnp.float32)
    excl = jax.lax.dot(mf, upper)
    ones_col = jnp.ones((t, 1), jnp.float32)
    rs_col = jax.lax.dot(mf, ones_col)
    rr = lax.broadcasted_iota(jnp.int32, (bt, bt), 0)
    cc = lax.broadcasted_iota(jnp.int32, (bt, bt), 1)
    blk = jnp.logical_and(cc < rr, (cc // t) == (rr // t)).astype(jnp.float32)
    row_off = jax.lax.dot(blk, rs_col)
    posf = excl + row_off
    posi = posf.astype(jnp.int32)
    pos_ref[...] = jnp.where(mask, posi, jnp.int32(CAP))


def _select(logit2):
    b, n = logit2.shape
    t = 128
    return pl.pallas_call(
        _select_body,
        out_shape=jax.ShapeDtypeStruct((b * t, t), jnp.int32),
    )(logit2.reshape(b * t, t))


def _compact(logit3, pos2):
    b_total, t, _ = logit3.shape
    n = t * t
    mesh = plsc.VectorSubcoreMesh(core_axis_name="c", subcore_axis_name="s")

    @functools.partial(
        pl.kernel,
        out_type=[jax.ShapeDtypeStruct((b_total, CAP), jnp.float32),
                  jax.ShapeDtypeStruct((b_total, CAP), jnp.int32)],
        mesh=mesh,
        compiler_params=pltpu.CompilerParams(needs_layout_passes=False),
        scratch_types=[
            pltpu.VMEM((t, t), jnp.float32),
            pltpu.VMEM((t, t), jnp.int32),
            pltpu.VMEM((CAPP,), jnp.float32),
            pltpu.VMEM((CAPP,), jnp.int32),
            pltpu.SemaphoreType.DMA,
        ],
    )
    def run(logit_hbm, pos_hbm, ckey_hbm, cidx_hbm, vals_v, pos_v, ckey_v,
            cidx_v, sem):
        c = lax.axis_index("c")
        s = lax.axis_index("s")

        @pl.when(c == 0)
        def _():
            b = s
            cp = pltpu.async_copy(pos_hbm.at[pl.ds(b * t, t)], pos_v, sem)
            pltpu.sync_copy(logit_hbm.at[b], vals_v)
            cp.wait()
            lane = lax.broadcasted_iota(jnp.int32, (16,), 0)

            def pre(i, carry):
                ckey_v[pl.ds(i * 16, 16)] = jnp.full((16,), -jnp.inf,
                                                     jnp.float32)
                cidx_v[pl.ds(i * 16, 16)] = jnp.full((16,), PAD_IDX,
                                                     jnp.int32)
                return carry

            lax.fori_loop(0, CAPP // 16, pre, 0)

            def body(i, carry):
                row = i >> 3
                col = (i & 7) * 16
                v = vals_v[row, pl.ds(col, 16)]
                p = pos_v[row, pl.ds(col, 16)]
                m = p < CAP
                plsc.store_scatter(ckey_v, [p], v, mask=m)
                plsc.store_scatter(cidx_v, [p], lane + i * 16, mask=m)
                return carry

            lax.fori_loop(0, n // 16, body, 0)
            pltpu.sync_copy(ckey_v.at[pl.ds(0, CAP)], ckey_hbm.at[b])
            pltpu.sync_copy(cidx_v.at[pl.ds(0, CAP)], cidx_hbm.at[b])

    return run(logit3, pos2)


def _rank_body(k_ref, i_ref, out_ref):
    kk = k_ref[...]
    ii = i_ref[...]
    for b in range(kk.shape[0]):
        krow = kk[b:b + 1, :]
        irow = ii[b:b + 1, :]
        kcol = jnp.transpose(krow)
        icol = jnp.transpose(irow)
        gt = kcol > krow
        tie = jnp.logical_and(kcol == krow, icol < irow)
        out_ref[b:b + 1, :] = jnp.sum(
            jnp.logical_or(gt, tie).astype(jnp.int32), axis=0, keepdims=True)


def _rank(ckey, cidx):
    b = ckey.shape[0]
    return pl.pallas_call(
        _rank_body,
        out_shape=jax.ShapeDtypeStruct((b, CAP), jnp.int32),
    )(ckey, cidx)


def _gather_all(cidx, rank, map_flat, off_rows, tmap_rows):
    b_total = cidx.shape[0]
    n = map_flat.shape[0] // b_total
    d = map_flat.shape[1]
    nm1 = b_total * n - 1
    mesh = plsc.VectorSubcoreMesh(core_axis_name="c", subcore_axis_name="s")

    @functools.partial(
        pl.kernel,
        out_type=[jax.ShapeDtypeStruct((b_total, K, d), jnp.float32),
                  jax.ShapeDtypeStruct((b_total, K, 2), jnp.int32),
                  jax.ShapeDtypeStruct((b_total, K, 2), jnp.float32),
                  jax.ShapeDtypeStruct((b_total, K), jnp.float32)],
        mesh=mesh,
        compiler_params=pltpu.CompilerParams(needs_layout_passes=False,
                                             use_tc_tiling_on_sc=True),
        scratch_types=[
            pltpu.VMEM((CAP,), jnp.int32),
            pltpu.VMEM((CAP,), jnp.int32),
            pltpu.VMEM((4, 64), jnp.int32),
            pltpu.VMEM((HK,), jnp.int32),
            pltpu.VMEM((4, 128), jnp.int32),
            pltpu.VMEM((2, 128), jnp.int32),
            pltpu.VMEM((64, d), jnp.float32),
            pltpu.VMEM((128, 128), jnp.float32),
            pltpu.VMEM((HK, 2), jnp.float32),
            pltpu.VMEM((HK,), jnp.float32),
            pltpu.VMEM((HK, 2), jnp.int32),
            pltpu.SemaphoreType.DMA,
        ],
    )
    def run(cidx_hbm, rank_hbm, map_hbm, offr_hbm, tmapr_hbm,
            prop_hbm, pred_hbm, off_hbm, score_hbm,
            cidx_v, rank_v, gmap_v, gflat_v, orow_v, trow_v,
            rows_map, rows_sm, off_v, sc_v, pred_v, sem):
        c = lax.axis_index("c")
        s = lax.axis_index("s")
        b = s
        lo = c * HK
        pltpu.sync_copy(cidx_hbm.at[b], cidx_v)
        pltpu.sync_copy(rank_hbm.at[b], rank_v)
        lane = lax.broadcasted_iota(jnp.int32, (16,), 0)
        zeros16 = jnp.zeros((16,), jnp.int32)
        ones16 = jnp.full((16,), 1, jnp.int32)

        def sbody(i, carry):
            rk = rank_v[pl.ds(i * 16, 16)]
            ix = cidx_v[pl.ds(i * 16, 16)]
            rrel = rk - lo
            m = jnp.logical_and(rrel >= 0, rrel < HK)
            rsafe = rrel & (HK - 1)
            g = (ix + b * n) & nm1
            plsc.store_scatter(gmap_v, [rsafe >> 6, rsafe & 63], g, mask=m)
            plsc.store_scatter(gflat_v, [rsafe], g, mask=m)
            jl2 = (rsafe & 63) * 2
            plsc.store_scatter(orow_v, [rsafe >> 6, jl2], (g >> 7) * 2,
                               mask=m)
            plsc.store_scatter(orow_v, [rsafe >> 6, jl2 + 1],
                               (g >> 7) * 2 + 1, mask=m)
            plsc.store_scatter(trow_v, [rsafe >> 7, rsafe & 127], g >> 7,
                               mask=m)
            return carry

        lax.fori_loop(0, CAP // 16, sbody, 0)

        for j in range(4):
            pltpu.async_copy(map_hbm.at[gmap_v.at[j]], rows_map, sem).wait()
            pltpu.sync_copy(rows_map, prop_hbm.at[b, pl.ds(lo + j * 64, 64)])

        for tchunk in range(4):
            pltpu.async_copy(offr_hbm.at[orow_v.at[tchunk]], rows_sm,
                             sem).wait()
            for i in range(4):
                base = tchunk * 64 + i * 16
                g16 = gflat_v[pl.ds(base, 16)]
                j16 = lane + i * 16
                jl2 = ((j16 * 2) & 127)
                col = g16 & 127
                o0 = plsc.load_gather(rows_sm, [jl2, col])
                o1 = plsc.load_gather(rows_sm, [jl2 + 1, col])
                ridx = j16 + tchunk * 64
                plsc.store_scatter(off_v, [ridx, zeros16], o0)
                plsc.store_scatter(off_v, [ridx, ones16], o1)

        for tchunk in range(2):
            pltpu.async_copy(tmapr_hbm.at[trow_v.at[tchunk]], rows_sm,
                             sem).wait()
            for i in range(8):
                base = tchunk * 128 + i * 16
                g16 = gflat_v[pl.ds(base, 16)]
                j16 = lane + i * 16
                sc_v[pl.ds(base, 16)] = plsc.load_gather(
                    rows_sm, [j16, g16 & 127])

        for i in range(HK // 16):
            g16 = gflat_v[pl.ds(i * 16, 16)]
            ii = g16 - b * n
            ridx = lane + i * 16
            plsc.store_scatter(pred_v, [ridx, zeros16], ii >> 7)
            plsc.store_scatter(pred_v, [ridx, ones16], (ii & 127) + 1)

        out_slice = pl.ds(lo, HK)
        pltpu.sync_copy(pred_v, pred_hbm.at[b, out_slice])
        pltpu.sync_copy(off_v, off_hbm.at[b, out_slice])
        pltpu.sync_copy(sc_v, score_hbm.at[b, out_slice])

    return run(cidx, rank, map_flat, off_rows, tmap_rows)


@jax.jit
def kernel(selection_logit, map2d, offset_gt, tmap):
    b, t, _ = selection_logit.shape
    n = t * t
    d = map2d.shape[-1]
    logit2 = selection_logit.reshape(b, n)
    pos = _select(logit2)
    ckey, cidx = _compact(selection_logit, pos)
    rank = _rank(ckey, cidx)
    off_rows = offset_gt.transpose(0, 1, 3, 2).reshape(b * t * 2, t)
    return tuple(_gather_all(cidx, rank,
                             map2d.reshape(b * n, d),
                             off_rows,
                             tmap.reshape(b * n // 128, 128)))


# speedup vs baseline: 3.1172x; 1.0348x over previous
import functools

import jax
import jax.numpy as jnp
from jax import lax
from jax.experimental import pallas as pl
from jax.experimental.pallas import tpu as pltpu
from jax.experimental.pallas import tpu_sc as plsc

K = 512
CAP = 640
CAPP = CAP + 16
HK = K // 2
I32_MIN = -(2 ** 31)
PAD_IDX = 1 << 29
MASK31 = 0x7FFFFFFF


def _monotone_key(bits):
    return bits ^ ((bits >> 31) & jnp.int32(MASK31))


def _select_body(x_ref, pos_ref):
    x = x_ref[...]
    bt, t = x.shape
    b = bt // t
    bits = lax.bitcast_convert_type(x, jnp.int32)
    key = _monotone_key(bits).reshape(b, t, t)

    def cnt_ge(th):
        return jnp.sum((key >= th).astype(jnp.int32), axis=(1, 2),
                       keepdims=True)

    zero = jnp.zeros((b, 1, 1), jnp.int32)
    imin = jnp.full((b, 1, 1), I32_MIN, jnp.int32)
    th = jnp.where(cnt_ge(zero) >= K, zero, imin)
    for bit in range(30, -1, -1):
        trial = th | jnp.int32(1 << bit)
        th = jnp.where(cnt_ge(trial) >= K, trial, th)

    mask = (key >= th).reshape(bt, t)
    mf = mask.astype(jnp.float32)
    r_i = lax.broadcasted_iota(jnp.int32, (t, t), 0)
    c_i = lax.broadcasted_iota(jnp.int32, (t, t), 1)
    upper = (r_i < c_i).astype(jnp.float32)
    excl = jax.lax.dot(mf, upper)
    ones_col = jnp.ones((t, 1), jnp.float32)
    rs_col = jax.lax.dot(mf, ones_col)
    rr = lax.broadcasted_iota(jnp.int32, (bt, bt), 0)
    cc = lax.broadcasted_iota(jnp.int32, (bt, bt), 1)
    blk = jnp.logical_and(cc < rr, (cc // t) == (rr // t)).astype(jnp.float32)
    row_off = jax.lax.dot(blk, rs_col)
    posf = excl + row_off
    posi = posf.astype(jnp.int32)
    pos_ref[...] = jnp.where(mask, posi, jnp.int32(CAP))


def _select(logit2):
    b, n = logit2.shape
    t = 128
    return pl.pallas_call(
        _select_body,
        out_shape=jax.ShapeDtypeStruct((b * t, t), jnp.int32),
    )(logit2.reshape(b * t, t))


def _compact(logit3, pos2):
    b_total, t, _ = logit3.shape
    n = t * t
    mesh = plsc.VectorSubcoreMesh(core_axis_name="c", subcore_axis_name="s")

    @functools.partial(
        pl.kernel,
        out_type=[jax.ShapeDtypeStruct((b_total, CAP), jnp.float32),
                  jax.ShapeDtypeStruct((b_total, CAP), jnp.int32)],
        mesh=mesh,
        compiler_params=pltpu.CompilerParams(needs_layout_passes=False),
        scratch_types=[
            pltpu.VMEM((t, t), jnp.float32),
            pltpu.VMEM((t, t), jnp.int32),
            pltpu.VMEM((CAPP,), jnp.float32),
            pltpu.VMEM((CAPP,), jnp.int32),
            pltpu.SemaphoreType.DMA,
        ],
    )
    def run(logit_hbm, pos_hbm, ckey_hbm, cidx_hbm, vals_v, pos_v, ckey_v,
            cidx_v, sem):
        c = lax.axis_index("c")
        s = lax.axis_index("s")

        @pl.when(c == 0)
        def _():
            b = s
            cp = pltpu.async_copy(pos_hbm.at[pl.ds(b * t, t)], pos_v, sem)
            pltpu.sync_copy(logit_hbm.at[b], vals_v)
            cp.wait()
            lane = lax.broadcasted_iota(jnp.int32, (16,), 0)

            def pre(i, carry):
                ckey_v[pl.ds(i * 16, 16)] = jnp.full((16,), -jnp.inf,
                                                     jnp.float32)
                cidx_v[pl.ds(i * 16, 16)] = jnp.full((16,), PAD_IDX,
                                                     jnp.int32)
                return carry

            lax.fori_loop(0, CAPP // 16, pre, 0)

            def body(i, carry):
                row = i >> 3
                col = (i & 7) * 16
                v = vals_v[row, pl.ds(col, 16)]
                p = pos_v[row, pl.ds(col, 16)]
                m = p < CAP
                plsc.store_scatter(ckey_v, [p], v, mask=m)
                plsc.store_scatter(cidx_v, [p], lane + i * 16, mask=m)
                return carry

            lax.fori_loop(0, n // 16, body, 0)
            pltpu.sync_copy(ckey_v.at[pl.ds(0, CAP)], ckey_hbm.at[b])
            pltpu.sync_copy(cidx_v.at[pl.ds(0, CAP)], cidx_hbm.at[b])

    return run(logit3, pos2)


def _rank_body(k_ref, i_ref, out_ref):
    kk = k_ref[...]
    ii = i_ref[...]
    for b in range(kk.shape[0]):
        krow = kk[b:b + 1, :]
        irow = ii[b:b + 1, :]
        kcol = jnp.transpose(krow)
        icol = jnp.transpose(irow)
        gt = kcol > krow
        tie = jnp.logical_and(kcol == krow, icol < irow)
        out_ref[b:b + 1, :] = jnp.sum(
            jnp.logical_or(gt, tie).astype(jnp.int32), axis=0, keepdims=True)


def _rank(ckey, cidx):
    b = ckey.shape[0]
    return pl.pallas_call(
        _rank_body,
        out_shape=jax.ShapeDtypeStruct((b, CAP), jnp.int32),
    )(ckey, cidx)


def _gather_all(cidx, rank, map_flat, off_rows, tmap_rows):
    b_total = cidx.shape[0]
    n = map_flat.shape[0] // b_total
    d = map_flat.shape[1]
    nm1 = b_total * n - 1
    mesh = plsc.VectorSubcoreMesh(core_axis_name="c", subcore_axis_name="s")

    @functools.partial(
        pl.kernel,
        out_type=[jax.ShapeDtypeStruct((b_total, K, d), jnp.float32),
                  jax.ShapeDtypeStruct((b_total, K, 2), jnp.int32),
                  jax.ShapeDtypeStruct((b_total, K, 2), jnp.float32),
                  jax.ShapeDtypeStruct((b_total, K), jnp.float32)],
        mesh=mesh,
        compiler_params=pltpu.CompilerParams(needs_layout_passes=False,
                                             use_tc_tiling_on_sc=True),
        scratch_types=[
            pltpu.VMEM((CAP,), jnp.int32),
            pltpu.VMEM((CAP,), jnp.int32),
            pltpu.VMEM((4, 64), jnp.int32),
            pltpu.VMEM((HK,), jnp.int32),
            pltpu.VMEM((4, 128), jnp.int32),
            pltpu.VMEM((2, 128), jnp.int32),
            pltpu.VMEM((64, d), jnp.float32),
            pltpu.VMEM((64, d), jnp.float32),
            pltpu.VMEM((128, 128), jnp.float32),
            pltpu.VMEM((HK, 2), jnp.float32),
            pltpu.VMEM((HK,), jnp.float32),
            pltpu.VMEM((HK, 2), jnp.int32),
            pltpu.SemaphoreType.DMA,
            pltpu.SemaphoreType.DMA,
            pltpu.SemaphoreType.DMA,
            pltpu.SemaphoreType.DMA,
        ],
    )
    def run(cidx_hbm, rank_hbm, map_hbm, offr_hbm, tmapr_hbm,
            prop_hbm, pred_hbm, off_hbm, score_hbm,
            cidx_v, rank_v, gmap_v, gflat_v, orow_v, trow_v,
            rows_map_a, rows_map_b, rows_sm,
            off_v, sc_v, pred_v, sem_a, sem_b, sem_c, sem_d):
        c = lax.axis_index("c")
        s = lax.axis_index("s")
        b = s
        lo = c * HK
        pltpu.sync_copy(cidx_hbm.at[b], cidx_v)
        pltpu.sync_copy(rank_hbm.at[b], rank_v)
        lane = lax.broadcasted_iota(jnp.int32, (16,), 0)
        zeros16 = jnp.zeros((16,), jnp.int32)
        ones16 = jnp.full((16,), 1, jnp.int32)

        def sbody(i, carry):
            rk = rank_v[pl.ds(i * 16, 16)]
            ix = cidx_v[pl.ds(i * 16, 16)]
            rrel = rk - lo
            m = jnp.logical_and(rrel >= 0, rrel < HK)
            rsafe = rrel & (HK - 1)
            g = (ix + b * n) & nm1
            plsc.store_scatter(gmap_v, [rsafe >> 6, rsafe & 63], g, mask=m)
            plsc.store_scatter(gflat_v, [rsafe], g, mask=m)
            jl2 = (rsafe & 63) * 2
            plsc.store_scatter(orow_v, [rsafe >> 6, jl2], (g >> 7) * 2,
                               mask=m)
            plsc.store_scatter(orow_v, [rsafe >> 6, jl2 + 1],
                               (g >> 7) * 2 + 1, mask=m)
            plsc.store_scatter(trow_v, [rsafe >> 7, rsafe & 127], g >> 7,
                               mask=m)
            return carry

        lax.fori_loop(0, CAP // 16, sbody, 0)

        map_bufs = [rows_map_a, rows_map_b]
        map_sems = [sem_a, sem_b]
        map_cps = [
            pltpu.async_copy(map_hbm.at[gmap_v.at[0]], rows_map_a, sem_a),
            pltpu.async_copy(map_hbm.at[gmap_v.at[1]], rows_map_b, sem_b),
        ]
        sm_cps = [
            pltpu.async_copy(offr_hbm.at[orow_v.at[0]], rows_sm, sem_c),
        ]
        for j in range(4):
            map_cps[j].wait()
            pltpu.sync_copy(map_bufs[j % 2],
                            prop_hbm.at[b, pl.ds(lo + j * 64, 64)])
            if j + 2 < 4:
                map_cps.append(pltpu.async_copy(
                    map_hbm.at[gmap_v.at[j + 2]], map_bufs[j % 2],
                    map_sems[j % 2]))

        for tchunk in range(4):
            buf = rows_sm
            sm_cps[tchunk].wait()
            for i in range(4):
                base = tchunk * 64 + i * 16
                g16 = gflat_v[pl.ds(base, 16)]
                j16 = lane + i * 16
                jl2 = ((j16 * 2) & 127)
                col = g16 & 127
                o0 = plsc.load_gather(buf, [jl2, col])
                o1 = plsc.load_gather(buf, [jl2 + 1, col])
                ridx = j16 + tchunk * 64
                plsc.store_scatter(off_v, [ridx, zeros16], o0)
                plsc.store_scatter(off_v, [ridx, ones16], o1)
            if tchunk + 1 < 4:
                sm_cps.append(pltpu.async_copy(
                    offr_hbm.at[orow_v.at[tchunk + 1]], buf, sem_c))
            else:
                sm_cps.append(pltpu.async_copy(
                    tmapr_hbm.at[trow_v.at[0]], buf, sem_c))

        for tchunk in range(2):
            buf = rows_sm
            sm_cps[4 + tchunk].wait()
            for i in range(8):
                base = tchunk * 128 + i * 16
                g16 = gflat_v[pl.ds(base, 16)]
                j16 = lane + i * 16
                sc_v[pl.ds(base, 16)] = plsc.load_gather(
                    buf, [j16, g16 & 127])
            if tchunk == 0:
                sm_cps.append(pltpu.async_copy(
                    tmapr_hbm.at[trow_v.at[1]], buf, sem_c))

        for i in range(HK // 16):
            g16 = gflat_v[pl.ds(i * 16, 16)]
            ii = g16 - b * n
            ridx = lane + i * 16
            plsc.store_scatter(pred_v, [ridx, zeros16], ii >> 7)
            plsc.store_scatter(pred_v, [ridx, ones16], (ii & 127) + 1)

        out_slice = pl.ds(lo, HK)
        pltpu.sync_copy(pred_v, pred_hbm.at[b, out_slice])
        pltpu.sync_copy(off_v, off_hbm.at[b, out_slice])
        pltpu.sync_copy(sc_v, score_hbm.at[b, out_slice])

    return run(cidx, rank, map_flat, off_rows, tmap_rows)


@jax.jit
def kernel(selection_logit, map2d, offset_gt, tmap):
    b, t, _ = selection_logit.shape
    n = t * t
    d = map2d.shape[-1]
    logit2 = selection_logit.reshape(b, n)
    pos = _select(logit2)
    ckey, cidx = _compact(selection_logit, pos)
    rank = _rank(ckey, cidx)
    off_rows = offset_gt.transpose(0, 1, 3, 2).reshape(b * t * 2, t)
    return tuple(_gather_all(cidx, rank,
                             map2d.reshape(b * n, d),
                             off_rows,
                             tmap.reshape(b * n // 128, 128)))
